# Initial kernel scaffold; baseline (speedup 1.0000x reference)
import jax
import jax.numpy as jnp
from jax.experimental import pallas as pl


def kernel(sender_input, labels):
    raise NotImplementedError("write your pallas kernel here")


# speedup vs baseline: 4.4741x; 4.4741x over previous
import jax
import jax.numpy as jnp
from jax.experimental import pallas as pl

N_ATTRIBUTES = 50
N_VALUES = 100
CURRICULUM_LEVEL = 25
ROWS = 256


def _threefry_mantissa(counts_lo):
    u32 = jnp.uint32
    ks0 = u32(0)
    ks1 = u32(42)
    ks2 = u32(0x1BD11BDA ^ 42)

    def rotl(x, r):
        return jax.lax.shift_left(x, u32(r)) | jax.lax.shift_right_logical(
            x, u32(32 - r))

    def four_rounds(x0, x1, rots):
        for r in rots:
            x0 = x0 + x1
            x1 = rotl(x1, r) ^ x0
        return x0, x1

    R0 = (13, 15, 26, 6)
    R1 = (17, 29, 16, 24)
    x0 = jnp.zeros_like(counts_lo) + ks0
    x1 = counts_lo + ks1
    x0, x1 = four_rounds(x0, x1, R0)
    x0 = x0 + ks1
    x1 = x1 + ks2 + u32(1)
    x0, x1 = four_rounds(x0, x1, R1)
    x0 = x0 + ks2
    x1 = x1 + ks0 + u32(2)
    x0, x1 = four_rounds(x0, x1, R0)
    x0 = x0 + ks0
    x1 = x1 + ks1 + u32(3)
    x0, x1 = four_rounds(x0, x1, R1)
    x0 = x0 + ks1
    x1 = x1 + ks2 + u32(4)
    x0, x1 = four_rounds(x0, x1, R0)
    x0 = x0 + ks2
    x1 = x1 + ks0 + u32(5)
    return jax.lax.shift_right_logical(x0 ^ x1, u32(9))


def _n_revealed_block(base_row, rows):
    cl = CURRICULUM_LEVEL
    r = jax.lax.broadcasted_iota(jnp.uint32, (rows, cl), 0)
    j = jax.lax.broadcasted_iota(jnp.uint32, (rows, cl), 1)
    counts = (base_row.astype(jnp.uint32) + r) * jnp.uint32(cl) + j
    m = _threefry_mantissa(counts).astype(jnp.int32)
    row_max = jnp.max(m, axis=1, keepdims=True)
    ji = jax.lax.broadcasted_iota(jnp.int32, (rows, cl), 1)
    win = jnp.min(jnp.where(m == row_max, ji, jnp.int32(cl)), axis=1,
                  keepdims=True)
    return win + 1


def _body(x_ref, masked_ref, mask_ref):
    i = pl.program_id(0)
    n_rev = _n_revealed_block(i * ROWS, ROWS)
    a = jax.lax.broadcasted_iota(jnp.int32, (ROWS, N_ATTRIBUTES), 1)
    mask_ref[...] = (a < n_rev).astype(jnp.float32)
    c = jax.lax.broadcasted_iota(jnp.int32, (ROWS, N_ATTRIBUTES * N_VALUES), 1)
    masked_ref[...] = jnp.where(c < n_rev * N_VALUES, x_ref[...], 0.0)


def _run(sender_input, interpret=False):
    batch, width = sender_input.shape
    grid = batch // ROWS
    return pl.pallas_call(
        _body,
        grid=(grid,),
        in_specs=[pl.BlockSpec((ROWS, width), lambda i: (i, 0))],
        out_specs=[
            pl.BlockSpec((ROWS, width), lambda i: (i, 0)),
            pl.BlockSpec((ROWS, N_ATTRIBUTES), lambda i: (i, 0)),
        ],
        out_shape=[
            jax.ShapeDtypeStruct((batch, width), jnp.float32),
            jax.ShapeDtypeStruct((batch, N_ATTRIBUTES), jnp.float32),
        ],
        interpret=interpret,
    )(sender_input)


def kernel(sender_input, labels):
    masked_input, mask = _run(sender_input)
    return masked_input, mask


# speedup vs baseline: 4.5095x; 1.0079x over previous
import jax
import jax.numpy as jnp
from jax.experimental import pallas as pl

N_ATTRIBUTES = 50
N_VALUES = 100
CURRICULUM_LEVEL = 25
ROWS = 512


def _threefry_mantissa(counts_lo):
    u32 = jnp.uint32
    ks0 = u32(0)
    ks1 = u32(42)
    ks2 = u32(0x1BD11BDA ^ 42)

    def rotl(x, r):
        return jax.lax.shift_left(x, u32(r)) | jax.lax.shift_right_logical(
            x, u32(32 - r))

    def four_rounds(x0, x1, rots):
        for r in rots:
            x0 = x0 + x1
            x1 = rotl(x1, r) ^ x0
        return x0, x1

    R0 = (13, 15, 26, 6)
    R1 = (17, 29, 16, 24)
    x0 = jnp.zeros_like(counts_lo) + ks0
    x1 = counts_lo + ks1
    x0, x1 = four_rounds(x0, x1, R0)
    x0 = x0 + ks1
    x1 = x1 + ks2 + u32(1)
    x0, x1 = four_rounds(x0, x1, R1)
    x0 = x0 + ks2
    x1 = x1 + ks0 + u32(2)
    x0, x1 = four_rounds(x0, x1, R0)
    x0 = x0 + ks0
    x1 = x1 + ks1 + u32(3)
    x0, x1 = four_rounds(x0, x1, R1)
    x0 = x0 + ks1
    x1 = x1 + ks2 + u32(4)
    x0, x1 = four_rounds(x0, x1, R0)
    x0 = x0 + ks2
    x1 = x1 + ks0 + u32(5)
    return jax.lax.shift_right_logical(x0 ^ x1, u32(9))


def _n_revealed_block(base_row, rows):
    cl = CURRICULUM_LEVEL
    r = jax.lax.broadcasted_iota(jnp.uint32, (rows, cl), 0)
    j = jax.lax.broadcasted_iota(jnp.uint32, (rows, cl), 1)
    counts = (base_row.astype(jnp.uint32) + r) * jnp.uint32(cl) + j
    m = _threefry_mantissa(counts).astype(jnp.int32)
    row_max = jnp.max(m, axis=1, keepdims=True)
    ji = jax.lax.broadcasted_iota(jnp.int32, (rows, cl), 1)
    win = jnp.min(jnp.where(m == row_max, ji, jnp.int32(cl)), axis=1,
                  keepdims=True)
    return win + 1


def _body(x_ref, masked_ref, mask_ref):
    i = pl.program_id(0)
    n_rev = _n_revealed_block(i * ROWS, ROWS)
    a = jax.lax.broadcasted_iota(jnp.int32, (ROWS, N_ATTRIBUTES), 1)
    mask_ref[...] = (a < n_rev).astype(jnp.float32)
    c = jax.lax.broadcasted_iota(jnp.int32, (ROWS, N_ATTRIBUTES * N_VALUES), 1)
    masked_ref[...] = jnp.where(c < n_rev * N_VALUES, x_ref[...], 0.0)


def _run(sender_input, interpret=False):
    batch, width = sender_input.shape
    grid = batch // ROWS
    return pl.pallas_call(
        _body,
        grid=(grid,),
        in_specs=[pl.BlockSpec((ROWS, width), lambda i: (i, 0))],
        out_specs=[
            pl.BlockSpec((ROWS, width), lambda i: (i, 0)),
            pl.BlockSpec((ROWS, N_ATTRIBUTES), lambda i: (i, 0)),
        ],
        out_shape=[
            jax.ShapeDtypeStruct((batch, width), jnp.float32),
            jax.ShapeDtypeStruct((batch, N_ATTRIBUTES), jnp.float32),
        ],
        interpret=interpret,
    )(sender_input)


def kernel(sender_input, labels):
    masked_input, mask = _run(sender_input)
    return masked_input, mask


# speedup vs baseline: 4.8074x; 1.0660x over previous
import jax
import jax.numpy as jnp
from jax.experimental import pallas as pl

N_ATTRIBUTES = 50
N_VALUES = 100
CURRICULUM_LEVEL = 25
ROWS = 512


def _threefry_mantissa(counts_lo):
    u32 = jnp.uint32
    ks0 = u32(0)
    ks1 = u32(42)
    ks2 = u32(0x1BD11BDA ^ 42)

    def rotl(x, r):
        return jax.lax.shift_left(x, u32(r)) | jax.lax.shift_right_logical(
            x, u32(32 - r))

    def four_rounds(x0, x1, rots):
        for r in rots:
            x0 = x0 + x1
            x1 = rotl(x1, r) ^ x0
        return x0, x1

    R0 = (13, 15, 26, 6)
    R1 = (17, 29, 16, 24)
    x0 = jnp.zeros_like(counts_lo) + ks0
    x1 = counts_lo + ks1
    x0, x1 = four_rounds(x0, x1, R0)
    x0 = x0 + ks1
    x1 = x1 + ks2 + u32(1)
    x0, x1 = four_rounds(x0, x1, R1)
    x0 = x0 + ks2
    x1 = x1 + ks0 + u32(2)
    x0, x1 = four_rounds(x0, x1, R0)
    x0 = x0 + ks0
    x1 = x1 + ks1 + u32(3)
    x0, x1 = four_rounds(x0, x1, R1)
    x0 = x0 + ks1
    x1 = x1 + ks2 + u32(4)
    x0, x1 = four_rounds(x0, x1, R0)
    x0 = x0 + ks2
    x1 = x1 + ks0 + u32(5)
    return jax.lax.shift_right_logical(x0 ^ x1, u32(9))


def _n_revealed_block(base_row, rows):
    cl = CURRICULUM_LEVEL
    r = jax.lax.broadcasted_iota(jnp.uint32, (rows, cl), 0)
    j = jax.lax.broadcasted_iota(jnp.uint32, (rows, cl), 1)
    counts = (base_row.astype(jnp.uint32) + r) * jnp.uint32(cl) + j
    m = _threefry_mantissa(counts).astype(jnp.int32)
    row_max = jnp.max(m, axis=1, keepdims=True)
    ji = jax.lax.broadcasted_iota(jnp.int32, (rows, cl), 1)
    win = jnp.min(jnp.where(m == row_max, ji, jnp.int32(cl)), axis=1,
                  keepdims=True)
    return win + 1


def _body(x_ref, masked_ref, mask_ref):
    i = pl.program_id(0)
    n_rev = _n_revealed_block(i * ROWS, ROWS)
    a = jax.lax.broadcasted_iota(jnp.int32, (ROWS, N_ATTRIBUTES), 1)
    mask_ref[...] = (a < n_rev).astype(jnp.float32)
    half = 2560
    c = jax.lax.broadcasted_iota(jnp.int32, (ROWS, half), 1)
    masked_ref[:, 0:half] = jnp.where(c < n_rev * N_VALUES, x_ref[...], 0.0)
    masked_ref[:, half:] = jnp.zeros((ROWS, N_ATTRIBUTES * N_VALUES - half),
                                     jnp.float32)


def _run(sender_input, interpret=False):
    batch, width = sender_input.shape
    grid = batch // ROWS
    return pl.pallas_call(
        _body,
        grid=(grid,),
        in_specs=[pl.BlockSpec((ROWS, 2560), lambda i: (i, 0))],
        out_specs=[
            pl.BlockSpec((ROWS, width), lambda i: (i, 0)),
            pl.BlockSpec((ROWS, N_ATTRIBUTES), lambda i: (i, 0)),
        ],
        out_shape=[
            jax.ShapeDtypeStruct((batch, width), jnp.float32),
            jax.ShapeDtypeStruct((batch, N_ATTRIBUTES), jnp.float32),
        ],
        interpret=interpret,
    )(sender_input)


def kernel(sender_input, labels):
    masked_input, mask = _run(sender_input)
    return masked_input, mask
